# baseline (speedup 1.0000x reference)
import numpy as np
import jax
import jax.numpy as jnp
from jax import lax
from jax.experimental import pallas as pl
from jax.experimental.pallas import tpu as pltpu
from jax.experimental.pallas import tpu_sc as plsc

_H = _W = 512
_B = 32
_NPI = 512
_IMGS = 8
_L = 16

_gen = np.random.default_rng(0)
_RX = int(_gen.integers(0, _W - 3 + 1))
_RY = int(_gen.integers(0, _H - 3 + 1))

_PAT = (((np.arange(_NPI)[:, None] >> np.arange(8, -1, -1)[None, :]) & 1)
        .astype(np.float32).T.copy())

_NOISE = np.asarray(
    jax.random.uniform(jax.random.key(42), (_B, _NPI), jnp.float32)) * 0.5

def _cell_sets():
    cells = []
    for i in range(1, 6):
        for j in range(1, 6):
            wsum = []
            psum = []
            for a in (i - 1, i, i + 1):
                for b in (j - 1, j, j + 1):
                    if 2 <= a <= 4 and 2 <= b <= 4:
                        psum.append(3 * (a - 2) + (b - 2))
                    else:
                        wsum.append(7 * a + b)
            if 2 <= i <= 4 and 2 <= j <= 4:
                cc = ("p", 3 * (i - 2) + (j - 2))
            else:
                cc = ("w", 7 * i + j)
            cells.append((wsum, psum, cc, 5 * (i - 1) + (j - 1)))
    return cells


_CELLS = _cell_sets()


def _gather(v, idx):
    return v.at[idx].get(mode="promise_in_bounds")


def _bfly(v, iota, op):
    for sh in (8, 4, 2, 1):
        v = op(v, _gather(v, iota ^ sh))
    return v


def _masked_scalar(vregs, iota, flat_idx):
    if not flat_idx:
        return 0.0
    if len(flat_idx) == 1:
        ch, lane = divmod(flat_idx[0], _L)
        return _gather(vregs[ch], iota * 0 + lane)
    tot = None
    by_chunk = {}
    for f in flat_idx:
        by_chunk.setdefault(f // _L, []).append(f % _L)
    for ch, lanes in by_chunk.items():
        m = None
        for lane in lanes:
            e = iota == lane
            m = e if m is None else (m | e)
        part = jnp.where(m, vregs[ch], 0.0)
        tot = part if tot is None else tot + part
    return _bfly(tot, iota, jnp.add)


def _sc_search_body(w_hbm, t_hbm, n_hbm, bits_hbm,
                    w_v, t_v, n_v, bits_v):
    nc = 2
    wid = lax.axis_index("s") * nc + lax.axis_index("c")
    pltpu.sync_copy(w_hbm.at[wid], w_v)
    pltpu.sync_copy(t_hbm.at[wid], t_v)
    pltpu.sync_copy(n_hbm.at[wid], n_v)

    iota = lax.iota(jnp.int32, _L)
    wr = [w_v[pl.ds(ch * _L, _L)] for ch in range(4)]
    tr = [t_v[pl.ds(ch * _L, _L)] for ch in range(2)]
    consts = []
    for wsum, psum, cc, tpos in _CELLS:
        ws = _masked_scalar(wr, iota, wsum)
        tv = _masked_scalar(tr, iota, [tpos])
        cw = None if cc[0] == "p" else _masked_scalar(wr, iota, [cc[1]])
        consts.append((ws, tv, cw))

    def chunk_body(c4, carry):
      for u in range(4):
        c = c4 * 4 + u
        best_val, best_idx = carry
        idxv = c * _L + iota
        pk = [((idxv >> (8 - k)) & 1).astype(jnp.float32) for k in range(9)]
        rowsums = []
        for r in range(3):
            a, b2, c3 = pk[3 * r], pk[3 * r + 1], pk[3 * r + 2]
            ab, bc = a + b2, b2 + c3
            rowsums.append({(0, 1): a, (1, 2): b2, (2, 3): c3,
                            (0, 2): ab, (1, 3): bc, (0, 3): ab + c3})
        err = None
        for (wsum, psum, cc, tpos), (ws, tv, cw) in zip(_CELLS, consts):
            rows = sorted(set(k // 3 for k in psum + ([cc[1]] if cc[0] == "p" else [])))
            cols = sorted(set(k % 3 for k in psum + ([cc[1]] if cc[0] == "p" else [])))
            span = (cols[0], cols[-1] + 1)
            ps = None
            for r in rows:
                v = rowsums[r][span]
                ps = v if ps is None else ps + v
            ssum = ps + ws
            cv = pk[cc[1]] if cc[0] == "p" else cw
            e = (jnp.clip(ssum - 2.0, 0.0, 1.0)
                 - jnp.clip(ssum - cv - 3.0, 0.0, 1.0))
            d = jnp.abs(e - tv)
            err = d if err is None else err + d
        seeded = err + n_v[pl.ds(c * _L, _L)]
        upd = seeded < best_val
        carry = (jnp.where(upd, seeded, best_val),
                 jnp.where(upd, idxv, best_idx))
      return carry

    best_val, best_idx = lax.fori_loop(
        0, _NPI // _L // 4,
        chunk_body,
        ((iota * 0).astype(jnp.float32) + 1e9,
         iota * 0))

    m = _bfly(best_val, iota, jnp.minimum)
    cand = jnp.where(best_val == m, best_idx, _NPI)
    bidx = _bfly(cand, iota, jnp.minimum)
    sh = jnp.maximum(8 - iota, 0)
    bits = jnp.where(iota < 9, (bidx >> sh) & 1, 0).astype(jnp.float32)
    bits_v[...] = bits
    pltpu.sync_copy(bits_v, bits_hbm.at[wid])


def _sc_search(wins64, tws32, noise):
    mesh = plsc.VectorSubcoreMesh(core_axis_name="c", subcore_axis_name="s")
    kfn = pl.kernel(
        _sc_search_body,
        out_type=jax.ShapeDtypeStruct((_B, _L), jnp.float32),
        mesh=mesh,
        scratch_types=[
            pltpu.VMEM((64,), jnp.float32),
            pltpu.VMEM((32,), jnp.float32),
            pltpu.VMEM((_NPI,), jnp.float32),
            pltpu.VMEM((_L,), jnp.float32),
        ],
    )
    return kfn(wins64, tws32, noise)


_BLO = (_RY // 8) * 8


def _copy_body(x_ref, o_ref):
    o_ref[...] = x_ref[...]


def _patch_body(band_ref, bits_ref, ob_ref):
    ob_ref[...] = band_ref[...]
    for m in range(_B):
        row = bits_ref[m, :]
        for i in range(3):
            ob_ref[m, 0, _RY - _BLO + i, _RX:_RX + 3] = row[3 * i:3 * i + 3]


def kernel(x, target):
    B = x.shape[0]
    wins = lax.slice(x, (0, 0, _RY - 2, _RX - 2),
                     (B, 1, _RY + 5, _RX + 5)).reshape(B, 49)
    wins64 = jnp.pad(wins, ((0, 0), (0, 15)))
    tws = lax.slice(target, (0, 0, _RY - 1, _RX - 1),
                    (B, 1, _RY + 4, _RX + 4)).reshape(B, 25)
    tws32 = jnp.pad(tws, ((0, 0), (0, 7)))
    bits = _sc_search(wins64, tws32, jnp.asarray(_NOISE))
    out1 = pl.pallas_call(
        _copy_body,
        grid=(B // _IMGS,),
        in_specs=[pl.BlockSpec((_IMGS, 1, _H, _W), lambda b: (b, 0, 0, 0))],
        out_specs=pl.BlockSpec((_IMGS, 1, _H, _W), lambda b: (b, 0, 0, 0)),
        out_shape=jax.ShapeDtypeStruct(x.shape, x.dtype),
        compiler_params=pltpu.CompilerParams(
            dimension_semantics=("arbitrary",)),
    )(x)
    out = pl.pallas_call(
        _patch_body,
        grid=(1,),
        in_specs=[
            pl.BlockSpec((_B, 1, 8, _W), lambda g: (0, 0, _BLO // 8, 0)),
            pl.BlockSpec((_B, _L), lambda g: (0, 0)),
        ],
        out_specs=pl.BlockSpec((_B, 1, 8, _W), lambda g: (0, 0, _BLO // 8, 0)),
        out_shape=jax.ShapeDtypeStruct(x.shape, x.dtype),
        input_output_aliases={0: 0},
    )(out1, bits)
    return out


# speedup vs baseline: 1.0782x; 1.0782x over previous
import numpy as np
import jax
import jax.numpy as jnp
from jax import lax
from jax.experimental import pallas as pl
from jax.experimental.pallas import tpu as pltpu
from jax.experimental.pallas import tpu_sc as plsc

_H = _W = 512
_B = 32
_NPI = 512
_IMGS = 8
_L = 16

_gen = np.random.default_rng(0)
_RX = int(_gen.integers(0, _W - 3 + 1))
_RY = int(_gen.integers(0, _H - 3 + 1))

_NOISE = np.asarray(
    jax.random.uniform(jax.random.key(42), (_B, _NPI), jnp.float32)) * 0.5

def _cell_sets():
    cells = []
    for i in range(1, 6):
        for j in range(1, 6):
            wsum = []
            psum = []
            for a in (i - 1, i, i + 1):
                for b in (j - 1, j, j + 1):
                    if 2 <= a <= 4 and 2 <= b <= 4:
                        psum.append(3 * (a - 2) + (b - 2))
                    else:
                        wsum.append((2 + a, 47 + b))
            if 2 <= i <= 4 and 2 <= j <= 4:
                cc = ("p", 3 * (i - 2) + (j - 2))
            else:
                cc = ("w", (2 + i, 47 + j))
            cells.append((wsum, psum, cc, (3 + (i - 1), 48 + (j - 1))))
    return cells


_CELLS = _cell_sets()


def _gather(v, idx):
    return v.at[idx].get(mode="promise_in_bounds")


def _bfly(v, iota, op):
    for sh in (8, 4, 2, 1):
        v = op(v, _gather(v, iota ^ sh))
    return v


def _masked_scalar(vregs, iota, idx_pairs):
    if not idx_pairs:
        return 0.0
    if len(idx_pairs) == 1:
        row, l128 = idx_pairs[0]
        return _gather(vregs[(row, l128 // _L)], iota * 0 + l128 % _L)
    tot = None
    by_chunk = {}
    for row, l128 in idx_pairs:
        by_chunk.setdefault((row, l128 // _L), []).append(l128 % _L)
    for ch, lanes in by_chunk.items():
        m = None
        for lane in lanes:
            e = iota == lane
            m = e if m is None else (m | e)
        part = jnp.where(m, vregs[ch], 0.0)
        tot = part if tot is None else tot + part
    return _bfly(tot, iota, jnp.add)


def _sc_search_body(x_hbm, t_hbm, n_hbm, bits_hbm,
                    w_v, t_v, n_v, bits_v):
    nc = 2
    wid = lax.axis_index("s") * nc + lax.axis_index("c")
    pltpu.sync_copy(
        x_hbm.at[wid, 0, pl.ds(320, 16), pl.ds(384, 128)], w_v)
    pltpu.sync_copy(
        t_hbm.at[wid, 0, pl.ds(320, 8), pl.ds(384, 128)], t_v)
    pltpu.sync_copy(n_hbm.at[wid], n_v)

    iota = lax.iota(jnp.int32, _L)
    wr = {(r, c): w_v[r, pl.ds(c * _L, _L)]
          for r in range(2, 9) for c in (2, 3)}
    tr = {(r, 3): t_v[r, pl.ds(48, _L)] for r in range(3, 8)}
    consts = []
    for wsum, psum, cc, tpos in _CELLS:
        ws = _masked_scalar(wr, iota, wsum)
        tv = _masked_scalar(tr, iota, [tpos])
        cw = None if cc[0] == "p" else _masked_scalar(wr, iota, [cc[1]])
        consts.append((ws, tv, cw))

    def chunk_body(c4, carry):
      for u in range(4):
        c = c4 * 4 + u
        best_val, best_idx = carry
        idxv = c * _L + iota
        pk = [((idxv >> (8 - k)) & 1).astype(jnp.float32) for k in range(9)]
        rowsums = []
        for r in range(3):
            a, b2, c3 = pk[3 * r], pk[3 * r + 1], pk[3 * r + 2]
            ab, bc = a + b2, b2 + c3
            rowsums.append({(0, 1): a, (1, 2): b2, (2, 3): c3,
                            (0, 2): ab, (1, 3): bc, (0, 3): ab + c3})
        err = None
        for (wsum, psum, cc, tpos), (ws, tv, cw) in zip(_CELLS, consts):
            rows = sorted(set(k // 3 for k in psum + ([cc[1]] if cc[0] == "p" else [])))
            cols = sorted(set(k % 3 for k in psum + ([cc[1]] if cc[0] == "p" else [])))
            span = (cols[0], cols[-1] + 1)
            ps = None
            for r in rows:
                v = rowsums[r][span]
                ps = v if ps is None else ps + v
            ssum = ps + ws
            cv = pk[cc[1]] if cc[0] == "p" else cw
            e = (jnp.clip(ssum - 2.0, 0.0, 1.0)
                 - jnp.clip(ssum - cv - 3.0, 0.0, 1.0))
            d = jnp.abs(e - tv)
            err = d if err is None else err + d
        seeded = err + n_v[pl.ds(c * _L, _L)]
        upd = seeded < best_val
        carry = (jnp.where(upd, seeded, best_val),
                 jnp.where(upd, idxv, best_idx))
      return carry

    best_val, best_idx = lax.fori_loop(
        0, _NPI // _L // 4,
        chunk_body,
        ((iota * 0).astype(jnp.float32) + 1e9,
         iota * 0))

    m = _bfly(best_val, iota, jnp.minimum)
    cand = jnp.where(best_val == m, best_idx, _NPI)
    bidx = _bfly(cand, iota, jnp.minimum)
    sh = jnp.maximum(8 - iota, 0)
    bits = jnp.where(iota < 9, (bidx >> sh) & 1, 0).astype(jnp.float32)
    bits_v[...] = bits
    pltpu.sync_copy(bits_v, bits_hbm.at[wid])


def _sc_search(x, target, noise):
    mesh = plsc.VectorSubcoreMesh(core_axis_name="c", subcore_axis_name="s")
    kfn = pl.kernel(
        _sc_search_body,
        out_type=jax.ShapeDtypeStruct((_B, _L), jnp.float32),
        mesh=mesh,
        scratch_types=[
            pltpu.VMEM((16, 128), jnp.float32),
            pltpu.VMEM((8, 128), jnp.float32),
            pltpu.VMEM((_NPI,), jnp.float32),
            pltpu.VMEM((_L,), jnp.float32),
        ],
    )
    return kfn(x, target, noise)


_BLO = (_RY // 8) * 8


def _copy_body(x_ref, o_ref):
    o_ref[...] = x_ref[...]


def _patch_body(band_ref, bits_ref, ob_ref):
    ob_ref[...] = band_ref[...]
    for m in range(_B):
        row = bits_ref[m, :]
        for i in range(3):
            ob_ref[m, 0, _RY - _BLO + i, _RX:_RX + 3] = row[3 * i:3 * i + 3]


def kernel(x, target):
    B = x.shape[0]
    bits = _sc_search(x, target, jnp.asarray(_NOISE))
    out1 = pl.pallas_call(
        _copy_body,
        grid=(B // _IMGS,),
        in_specs=[pl.BlockSpec((_IMGS, 1, _H, _W), lambda b: (b, 0, 0, 0))],
        out_specs=pl.BlockSpec((_IMGS, 1, _H, _W), lambda b: (b, 0, 0, 0)),
        out_shape=jax.ShapeDtypeStruct(x.shape, x.dtype),
        compiler_params=pltpu.CompilerParams(
            dimension_semantics=("arbitrary",)),
    )(x)
    out = pl.pallas_call(
        _patch_body,
        grid=(1,),
        in_specs=[
            pl.BlockSpec((_B, 1, 8, _W), lambda g: (0, 0, _BLO // 8, 0)),
            pl.BlockSpec((_B, _L), lambda g: (0, 0)),
        ],
        out_specs=pl.BlockSpec((_B, 1, 8, _W), lambda g: (0, 0, _BLO // 8, 0)),
        out_shape=jax.ShapeDtypeStruct(x.shape, x.dtype),
        input_output_aliases={0: 0},
    )(out1, bits)
    return out
